# Initial kernel scaffold; baseline (speedup 1.0000x reference)
import jax
import jax.numpy as jnp
from jax.experimental import pallas as pl


def kernel(x, rw1, action, edge_index, net_W, net_b, gcn1_W, gcn1_b, hv_W, hv_b, gcn2_W, gcn2_b):
    raise NotImplementedError("write your pallas kernel here")


# speedup vs baseline: 7.1581x; 7.1581x over previous
import functools

import jax
import jax.numpy as jnp
from jax import lax
from jax.experimental import pallas as pl
from jax.experimental.pallas import tpu as pltpu
from jax.experimental.pallas import tpu_sc as plsc

N = 10000
E = 320000
NC, NS, LANES = 2, 16, 16
NW = NC * NS
NPAD = 10240
EPT = 20480
EPAD = EPT * NS
KE = 128
RPT = NPAD // NS
HB = 1024
GRID = NPAD // HB

_HCH = 2000
_EPW = E // NW


def _hist_kernel(dst):
    mesh = plsc.VectorSubcoreMesh(core_axis_name="c", subcore_axis_name="s")

    @functools.partial(
        pl.kernel,
        out_type=jax.ShapeDtypeStruct((NW, NPAD), jnp.float32),
        mesh=mesh,
        scratch_types=[
            pltpu.VMEM((_HCH,), jnp.int32),
            pltpu.VMEM((NPAD,), jnp.float32),
        ],
        compiler_params=pltpu.CompilerParams(needs_layout_passes=False),
    )
    def hist(dst_hbm, out_hbm, dbuf, hacc):
        c = lax.axis_index("c")
        s = lax.axis_index("s")
        wid = s * NC + c
        zeros = jnp.zeros((LANES,), jnp.float32)
        ones = jnp.ones((LANES,), jnp.float32)

        def zbody(j, carry):
            hacc[pl.ds(j * LANES, LANES)] = zeros
            return carry

        lax.fori_loop(0, NPAD // LANES, zbody, 0)

        for outer in range(_EPW // _HCH):
            base = wid * _EPW + outer * _HCH
            pltpu.sync_copy(dst_hbm.at[pl.ds(base, _HCH)], dbuf)

            def sbody(i, carry):
                v = dbuf[pl.ds(i * LANES, LANES)]
                plsc.addupdate_scatter(hacc, [v], ones)
                return carry

            lax.fori_loop(0, _HCH // LANES, sbody, 0)

        pltpu.sync_copy(hacc, out_hbm.at[wid])

    return hist(dst)


def _make_agg(width):
    mesh = plsc.VectorSubcoreMesh(core_axis_name="c", subcore_axis_name="s")

    @functools.partial(
        pl.kernel,
        out_type=jax.ShapeDtypeStruct((NC, NPAD, width), jnp.float32),
        mesh=mesh,
        scratch_types=[
            pltpu.VMEM_SHARED((NPAD, width), jnp.float32),
            pltpu.VMEM((KE,), jnp.int32),
            pltpu.VMEM((KE,), jnp.int32),
            pltpu.VMEM((KE, width), jnp.float32),
            pltpu.SemaphoreType.DMA,
        ],
        compiler_params=pltpu.CompilerParams(use_tc_tiling_on_sc=False),
    )
    def agg(ytab, src2, dstp, out, acc, sidx, didx, rows, sem):
        c = lax.axis_index("c")
        s = lax.axis_index("s")
        r0 = s * RPT
        pltpu.sync_copy(ytab.at[pl.ds(c * NPAD + r0, RPT)],
                        acc.at[pl.ds(r0, RPT)])
        plsc.subcore_barrier()

        e0 = s * EPT

        def step(t, carry):
            off = e0 + t * KE
            pltpu.sync_copy(src2.at[c, pl.ds(off, KE)], sidx)
            pltpu.sync_copy(dstp.at[pl.ds(off, KE)], didx)
            pltpu.async_copy(ytab.at[sidx], rows, sem).wait()
            pltpu.sync_copy(rows, acc.at[didx], add=True)
            return carry

        lax.fori_loop(0, EPT // KE, step, 0)
        plsc.subcore_barrier()
        pltpu.sync_copy(acc.at[pl.ds(r0, RPT)],
                        out.at[c, pl.ds(r0, RPT)])

    return agg


_agg128 = _make_agg(128)
_agg80 = _make_agg(80)


def _tc1(xp, rwp, acp, hist, net_W, net_b, g1Wh, g1Wr, g1Wa):
    def body(x_ref, rw_ref, ac_ref, h_ref, nw_ref, nb_ref, wh_ref, wr_ref,
             wa_ref, y1_ref):
        h = jnp.maximum(x_ref[...] @ nw_ref[...] + nb_ref[...], 0.0)
        xw = h @ wh_ref[...] + rw_ref[...] * wr_ref[...] + ac_ref[...] * wa_ref[...]
        deg = 1.0 + jnp.sum(h_ref[...], axis=0)
        dis = 1.0 / jnp.sqrt(deg)
        y = xw * dis[:, None]
        y1_ref[0] = y[:, :128]
        y1_ref[1] = y[:, 128:]

    return pl.pallas_call(
        body,
        grid=(GRID,),
        in_specs=[
            pl.BlockSpec((HB, 128), lambda i: (i, 0)),
            pl.BlockSpec((HB, 1), lambda i: (i, 0)),
            pl.BlockSpec((HB, 1), lambda i: (i, 0)),
            pl.BlockSpec((NW, HB), lambda i: (0, i)),
            pl.BlockSpec((128, 128), lambda i: (0, 0)),
            pl.BlockSpec((1, 128), lambda i: (0, 0)),
            pl.BlockSpec((128, 256), lambda i: (0, 0)),
            pl.BlockSpec((1, 256), lambda i: (0, 0)),
            pl.BlockSpec((1, 256), lambda i: (0, 0)),
        ],
        out_specs=pl.BlockSpec((2, HB, 128), lambda i: (0, i, 0)),
        out_shape=jax.ShapeDtypeStruct((2, NPAD, 128), jnp.float32),
    )(xp, rwp, acp, hist, net_W, net_b, g1Wh, g1Wr, g1Wa)


def _tc2(agg1, hist, g1b, hv_W, hv_b, epsp, g2Wp):
    def body(a_ref, h_ref, b1_ref, hw_ref, hb_ref, e_ref, w2_ref,
             y2_ref):
        deg = 1.0 + jnp.sum(h_ref[...], axis=0)
        dis = 1.0 / jnp.sqrt(deg)
        agg = jnp.concatenate([a_ref[0], a_ref[1]], axis=1)
        out1 = jnp.maximum(agg * dis[:, None] + b1_ref[...], 0.0)
        x3 = jnp.maximum(out1 @ hw_ref[...] + hb_ref[...], 0.0)
        mean, logvar = x3[:, :128], x3[:, 128:]
        hidden = mean + e_ref[...] * jnp.exp(jnp.clip(logvar, -5.0, 5.0))
        y2 = (hidden @ w2_ref[...]) * dis[:, None]
        y2_ref[0] = y2[:, :80]
        y2_ref[1] = y2[:, 80:]

    return pl.pallas_call(
        body,
        grid=(GRID,),
        in_specs=[
            pl.BlockSpec((2, HB, 128), lambda i: (0, i, 0)),
            pl.BlockSpec((NW, HB), lambda i: (0, i)),
            pl.BlockSpec((1, 256), lambda i: (0, 0)),
            pl.BlockSpec((256, 256), lambda i: (0, 0)),
            pl.BlockSpec((1, 256), lambda i: (0, 0)),
            pl.BlockSpec((HB, 128), lambda i: (i, 0)),
            pl.BlockSpec((128, 160), lambda i: (0, 0)),
        ],
        out_specs=pl.BlockSpec((2, HB, 80), lambda i: (0, i, 0)),
        out_shape=jax.ShapeDtypeStruct((2, NPAD, 80), jnp.float32),
    )(agg1, hist, g1b, hv_W, hv_b, epsp, g2Wp)


def _tc3(agg2, hist, g2b):
    def body(a_ref, h_ref, b2_ref, o_ref):
        deg = 1.0 + jnp.sum(h_ref[...], axis=0)
        dis = 1.0 / jnp.sqrt(deg)
        agg = jnp.concatenate([a_ref[0], a_ref[1]], axis=1)
        o_ref[...] = jnp.maximum(agg * dis[:, None] + b2_ref[...], 0.0)

    return pl.pallas_call(
        body,
        grid=(GRID,),
        in_specs=[
            pl.BlockSpec((2, HB, 80), lambda i: (0, i, 0)),
            pl.BlockSpec((NW, HB), lambda i: (0, i)),
            pl.BlockSpec((1, 160), lambda i: (0, 0)),
        ],
        out_specs=pl.BlockSpec((HB, 160), lambda i: (i, 0)),
        out_shape=jax.ShapeDtypeStruct((NPAD, 160), jnp.float32),
    )(agg2, hist, g2b)


def kernel(x, rw1, action, edge_index, net_W, net_b, gcn1_W, gcn1_b, hv_W,
           hv_b, gcn2_W, gcn2_b):
    src, dst = edge_index[0], edge_index[1]
    pad = jnp.full((EPAD - E,), N, dtype=src.dtype)
    srcp = jnp.concatenate([src, pad])
    dstp = jnp.concatenate([dst, pad])
    src2 = jnp.stack([srcp, srcp + NPAD])

    xp = jnp.pad(x, ((0, NPAD - N), (0, 0)))
    rwp = jnp.pad(rw1, ((0, NPAD - N), (0, 0)))
    acp = jnp.pad(action, ((0, NPAD - N), (0, 0)))
    eps = jax.random.normal(jax.random.key(42), (N, 128), dtype=jnp.float32)
    epsp = jnp.pad(eps, ((0, NPAD - N), (0, 0)))

    g1Wh = gcn1_W[:128]
    g1Wr = gcn1_W[128:129]
    g1Wa = gcn1_W[129:130]
    g2Wp = jnp.pad(gcn2_W, ((0, 0), (0, 31)))
    g2bp = jnp.pad(gcn2_b, (0, 31))

    hist = _hist_kernel(dst)
    y1tab = _tc1(xp, rwp, acp, hist, net_W, net_b[None, :], g1Wh, g1Wr, g1Wa)
    agg1 = _agg128(y1tab.reshape(2 * NPAD, 128), src2, dstp)
    y2tab = _tc2(agg1, hist, gcn1_b[None, :], hv_W, hv_b[None, :],
                 epsp, g2Wp)
    agg2 = _agg80(y2tab.reshape(2 * NPAD, 80), src2, dstp)
    res = _tc3(agg2, hist, g2bp[None, :])
    r = res[:N]
    return (r[:, :128], r[:, 128])


# speedup vs baseline: 10.2469x; 1.4315x over previous
import functools

import jax
import jax.numpy as jnp
from jax import lax
from jax.experimental import pallas as pl
from jax.experimental.pallas import tpu as pltpu
from jax.experimental.pallas import tpu_sc as plsc

N = 10000
E = 320000
NC, NS, LANES = 2, 16, 16
NW = NC * NS
NPAD = 10240
EPT = 20480
EPAD = EPT * NS
KE = 128
RPT = NPAD // NS
HB = 1024
GRID = NPAD // HB

_HCH = 2000
_EPW = E // NW


def _hist_kernel(dst):
    mesh = plsc.VectorSubcoreMesh(core_axis_name="c", subcore_axis_name="s")

    @functools.partial(
        pl.kernel,
        out_type=jax.ShapeDtypeStruct((NW, NPAD), jnp.float32),
        mesh=mesh,
        scratch_types=[
            pltpu.VMEM((_HCH,), jnp.int32),
            pltpu.VMEM((NPAD,), jnp.float32),
        ],
        compiler_params=pltpu.CompilerParams(needs_layout_passes=False),
    )
    def hist(dst_hbm, out_hbm, dbuf, hacc):
        c = lax.axis_index("c")
        s = lax.axis_index("s")
        wid = s * NC + c
        zeros = jnp.zeros((LANES,), jnp.float32)
        ones = jnp.ones((LANES,), jnp.float32)

        def zbody(j, carry):
            hacc[pl.ds(j * LANES, LANES)] = zeros
            return carry

        lax.fori_loop(0, NPAD // LANES, zbody, 0)

        for outer in range(_EPW // _HCH):
            base = wid * _EPW + outer * _HCH
            pltpu.sync_copy(dst_hbm.at[pl.ds(base, _HCH)], dbuf)

            def sbody(i, carry):
                v = dbuf[pl.ds(i * LANES, LANES)]
                plsc.addupdate_scatter(hacc, [v], ones)
                return carry

            lax.fori_loop(0, _HCH // LANES, sbody, 0)

        pltpu.sync_copy(hacc, out_hbm.at[wid])

    return hist(dst)


CHUNKS = EPT // KE
IDXB = 8
BLOCKS = CHUNKS // IDXB


def _make_agg(width):
    mesh = plsc.VectorSubcoreMesh(core_axis_name="c", subcore_axis_name="s")

    @functools.partial(
        pl.kernel,
        out_type=jax.ShapeDtypeStruct((NC, NPAD, width), jnp.float32),
        mesh=mesh,
        scratch_types=[
            pltpu.VMEM_SHARED((NPAD, width), jnp.float32),
            pltpu.VMEM((2, IDXB, KE), jnp.int32),
            pltpu.VMEM((2, IDXB, KE), jnp.int32),
            pltpu.VMEM((2, KE, width), jnp.float32),
            pltpu.SemaphoreType.DMA,
            pltpu.SemaphoreType.DMA,
            pltpu.SemaphoreType.DMA,
            pltpu.SemaphoreType.DMA,
        ],
        compiler_params=pltpu.CompilerParams(use_tc_tiling_on_sc=False),
    )
    def agg(ytab, src2, dstp, out, acc, sall, dall, rows, g0, g1, i0, i1):
        gsems = (g0, g1)
        isems = (i0, i1)
        c = lax.axis_index("c")
        s = lax.axis_index("s")
        r0 = s * RPT
        last = BLOCKS - 1

        pltpu.sync_copy(src2.at[c, s, 0], sall.at[0])
        pltpu.sync_copy(dstp.at[s, 0], dall.at[0])
        pltpu.async_copy(src2.at[c, s, 1], sall.at[1], isems[1])
        pltpu.async_copy(dstp.at[s, 1], dall.at[1], isems[1])
        pltpu.sync_copy(ytab.at[pl.ds(c * NPAD + r0, RPT)],
                        acc.at[pl.ds(r0, RPT)])
        plsc.subcore_barrier()

        for b in range(2):
            pltpu.async_copy(ytab.at[sall.at[0, b]], rows.at[b], gsems[b])

        def wait_idx(slot):
            pltpu.make_async_copy(src2.at[c, s, 0], sall.at[slot],
                                  isems[slot]).wait()
            pltpu.make_async_copy(dstp.at[s, 0], dall.at[slot],
                                  isems[slot]).wait()

        def block(slot, other):
            for j in range(IDXB):
                b = j % 2
                pltpu.make_async_copy(ytab.at[sall.at[slot, j]], rows.at[b],
                                      gsems[b]).wait()
                pltpu.sync_copy(rows.at[b], acc.at[dall.at[slot, j]],
                                add=True)
                if j == IDXB - 2:
                    wait_idx(other)
                jj = j + 2
                if jj < IDXB:
                    idx = sall.at[slot, jj]
                else:
                    idx = sall.at[other, jj - IDXB]
                pltpu.async_copy(ytab.at[idx], rows.at[b], gsems[b])

        def pair(i, carry):
            block(0, 1)
            nxt = jnp.minimum(2 * i + 2, last)
            pltpu.async_copy(src2.at[c, s, nxt], sall.at[0], isems[0])
            pltpu.async_copy(dstp.at[s, nxt], dall.at[0], isems[0])
            block(1, 0)
            nxt2 = jnp.minimum(2 * i + 3, last)
            pltpu.async_copy(src2.at[c, s, nxt2], sall.at[1], isems[1])
            pltpu.async_copy(dstp.at[s, nxt2], dall.at[1], isems[1])
            return carry

        lax.fori_loop(0, BLOCKS // 2, pair, 0)

        for b in range(2):
            pltpu.make_async_copy(ytab.at[sall.at[0, b]], rows.at[b],
                                  gsems[b]).wait()
        wait_idx(1)

        plsc.subcore_barrier()
        pltpu.sync_copy(acc.at[pl.ds(r0, RPT)],
                        out.at[c, pl.ds(r0, RPT)])

    return agg


_agg128 = _make_agg(128)
_agg80 = _make_agg(80)


def _tc1(xp, rwp, acp, hist, net_W, net_b, g1Wh, g1Wr, g1Wa):
    def body(x_ref, rw_ref, ac_ref, h_ref, nw_ref, nb_ref, wh_ref, wr_ref,
             wa_ref, y1_ref):
        h = jnp.maximum(x_ref[...] @ nw_ref[...] + nb_ref[...], 0.0)
        xw = h @ wh_ref[...] + rw_ref[...] * wr_ref[...] + ac_ref[...] * wa_ref[...]
        deg = 1.0 + jnp.sum(h_ref[...], axis=0)
        dis = 1.0 / jnp.sqrt(deg)
        y = xw * dis[:, None]
        y1_ref[0] = y[:, :128]
        y1_ref[1] = y[:, 128:]

    return pl.pallas_call(
        body,
        grid=(GRID,),
        in_specs=[
            pl.BlockSpec((HB, 128), lambda i: (i, 0)),
            pl.BlockSpec((HB, 1), lambda i: (i, 0)),
            pl.BlockSpec((HB, 1), lambda i: (i, 0)),
            pl.BlockSpec((NW, HB), lambda i: (0, i)),
            pl.BlockSpec((128, 128), lambda i: (0, 0)),
            pl.BlockSpec((1, 128), lambda i: (0, 0)),
            pl.BlockSpec((128, 256), lambda i: (0, 0)),
            pl.BlockSpec((1, 256), lambda i: (0, 0)),
            pl.BlockSpec((1, 256), lambda i: (0, 0)),
        ],
        out_specs=pl.BlockSpec((2, HB, 128), lambda i: (0, i, 0)),
        out_shape=jax.ShapeDtypeStruct((2, NPAD, 128), jnp.float32),
    )(xp, rwp, acp, hist, net_W, net_b, g1Wh, g1Wr, g1Wa)


def _tc2(agg1, hist, g1b, hv_W, hv_b, epsp, g2Wp):
    def body(a_ref, h_ref, b1_ref, hw_ref, hb_ref, e_ref, w2_ref,
             y2_ref):
        deg = 1.0 + jnp.sum(h_ref[...], axis=0)
        dis = 1.0 / jnp.sqrt(deg)
        agg = jnp.concatenate([a_ref[0], a_ref[1]], axis=1)
        out1 = jnp.maximum(agg * dis[:, None] + b1_ref[...], 0.0)
        x3 = jnp.maximum(out1 @ hw_ref[...] + hb_ref[...], 0.0)
        mean, logvar = x3[:, :128], x3[:, 128:]
        hidden = mean + e_ref[...] * jnp.exp(jnp.clip(logvar, -5.0, 5.0))
        y2 = (hidden @ w2_ref[...]) * dis[:, None]
        y2_ref[0] = y2[:, :80]
        y2_ref[1] = y2[:, 80:]

    return pl.pallas_call(
        body,
        grid=(GRID,),
        in_specs=[
            pl.BlockSpec((2, HB, 128), lambda i: (0, i, 0)),
            pl.BlockSpec((NW, HB), lambda i: (0, i)),
            pl.BlockSpec((1, 256), lambda i: (0, 0)),
            pl.BlockSpec((256, 256), lambda i: (0, 0)),
            pl.BlockSpec((1, 256), lambda i: (0, 0)),
            pl.BlockSpec((HB, 128), lambda i: (i, 0)),
            pl.BlockSpec((128, 160), lambda i: (0, 0)),
        ],
        out_specs=pl.BlockSpec((2, HB, 80), lambda i: (0, i, 0)),
        out_shape=jax.ShapeDtypeStruct((2, NPAD, 80), jnp.float32),
    )(agg1, hist, g1b, hv_W, hv_b, epsp, g2Wp)


def _tc3(agg2, hist, g2b):
    def body(a_ref, h_ref, b2_ref, o_ref):
        deg = 1.0 + jnp.sum(h_ref[...], axis=0)
        dis = 1.0 / jnp.sqrt(deg)
        agg = jnp.concatenate([a_ref[0], a_ref[1]], axis=1)
        o_ref[...] = jnp.maximum(agg * dis[:, None] + b2_ref[...], 0.0)

    return pl.pallas_call(
        body,
        grid=(GRID,),
        in_specs=[
            pl.BlockSpec((2, HB, 80), lambda i: (0, i, 0)),
            pl.BlockSpec((NW, HB), lambda i: (0, i)),
            pl.BlockSpec((1, 160), lambda i: (0, 0)),
        ],
        out_specs=pl.BlockSpec((HB, 160), lambda i: (i, 0)),
        out_shape=jax.ShapeDtypeStruct((NPAD, 160), jnp.float32),
    )(agg2, hist, g2b)


def kernel(x, rw1, action, edge_index, net_W, net_b, gcn1_W, gcn1_b, hv_W,
           hv_b, gcn2_W, gcn2_b):
    src, dst = edge_index[0], edge_index[1]
    pad = jnp.full((EPAD - E,), N, dtype=src.dtype)
    srcp = jnp.concatenate([src, pad])
    dstp = jnp.concatenate([dst, pad]).reshape(NS, BLOCKS, IDXB, KE)
    src2 = jnp.stack([srcp, srcp + NPAD]).reshape(NC, NS, BLOCKS, IDXB, KE)

    xp = jnp.pad(x, ((0, NPAD - N), (0, 0)))
    rwp = jnp.pad(rw1, ((0, NPAD - N), (0, 0)))
    acp = jnp.pad(action, ((0, NPAD - N), (0, 0)))
    eps = jax.random.normal(jax.random.key(42), (N, 128), dtype=jnp.float32)
    epsp = jnp.pad(eps, ((0, NPAD - N), (0, 0)))

    g1Wh = gcn1_W[:128]
    g1Wr = gcn1_W[128:129]
    g1Wa = gcn1_W[129:130]
    g2Wp = jnp.pad(gcn2_W, ((0, 0), (0, 31)))
    g2bp = jnp.pad(gcn2_b, (0, 31))

    hist = _hist_kernel(dst)
    y1tab = _tc1(xp, rwp, acp, hist, net_W, net_b[None, :], g1Wh, g1Wr, g1Wa)
    agg1 = _agg128(y1tab.reshape(2 * NPAD, 128), src2, dstp)
    y2tab = _tc2(agg1, hist, gcn1_b[None, :], hv_W, hv_b[None, :],
                 epsp, g2Wp)
    agg2 = _agg80(y2tab.reshape(2 * NPAD, 80), src2, dstp)
    res = _tc3(agg2, hist, g2bp[None, :])
    r = res[:N]
    return (r[:, :128], r[:, 128])


# speedup vs baseline: 18.3264x; 1.7885x over previous
import functools

import jax
import jax.numpy as jnp
from jax import lax
from jax.experimental import pallas as pl
from jax.experimental.pallas import tpu as pltpu
from jax.experimental.pallas import tpu_sc as plsc

N = 10000
E = 320000
NC, NS, LANES = 2, 16, 16
NW = NC * NS
NPAD = 10240
EPT = 20480
EPAD = EPT * NS
KE = 128
RPT = NPAD // NS
HB = 1024
GRID = NPAD // HB

_HCH = 2000
_EPW = E // NW


def _hist_kernel(dst):
    mesh = plsc.VectorSubcoreMesh(core_axis_name="c", subcore_axis_name="s")

    @functools.partial(
        pl.kernel,
        out_type=jax.ShapeDtypeStruct((NW, NPAD), jnp.float32),
        mesh=mesh,
        scratch_types=[
            pltpu.VMEM((_HCH,), jnp.int32),
            pltpu.VMEM((NPAD,), jnp.float32),
        ],
        compiler_params=pltpu.CompilerParams(needs_layout_passes=False),
    )
    def hist(dst_hbm, out_hbm, dbuf, hacc):
        c = lax.axis_index("c")
        s = lax.axis_index("s")
        wid = s * NC + c
        zeros = jnp.zeros((LANES,), jnp.float32)
        ones = jnp.ones((LANES,), jnp.float32)

        def zbody(j, carry):
            hacc[pl.ds(j * LANES, LANES)] = zeros
            return carry

        lax.fori_loop(0, NPAD // LANES, zbody, 0)

        for outer in range(_EPW // _HCH):
            base = wid * _EPW + outer * _HCH
            pltpu.sync_copy(dst_hbm.at[pl.ds(base, _HCH)], dbuf)

            def sbody(i, carry):
                v = dbuf[pl.ds(i * LANES, LANES)]
                plsc.addupdate_scatter(hacc, [v], ones)
                return carry

            lax.fori_loop(0, _HCH // LANES, sbody, 0)

        pltpu.sync_copy(hacc, out_hbm.at[wid])

    return hist(dst)


CHUNKS = EPT // KE
IDXB = 8
BLOCKS = CHUNKS // IDXB


def _make_agg(width, passes):
    mesh = plsc.VectorSubcoreMesh(core_axis_name="c", subcore_axis_name="s")

    @functools.partial(
        pl.kernel,
        out_type=jax.ShapeDtypeStruct((NC * passes, NPAD, width),
                                      jnp.float32),
        mesh=mesh,
        scratch_types=[
            pltpu.VMEM_SHARED((NPAD, width), jnp.float32),
            pltpu.VMEM_SHARED((NPAD, width), jnp.float32),
            pltpu.VMEM((2, IDXB, KE), jnp.int32),
            pltpu.VMEM((2, IDXB, KE), jnp.int32),
            pltpu.VMEM((2, KE, width), jnp.float32),
            pltpu.SemaphoreType.DMA,
            pltpu.SemaphoreType.DMA,
            pltpu.SemaphoreType.DMA,
            pltpu.SemaphoreType.DMA,
        ],
        compiler_params=pltpu.CompilerParams(use_tc_tiling_on_sc=False),
    )
    def agg(ytab, srcb, dstb, out, table, acc, sall, dall, rows,
            g0, g1, i0, i1):
        gsems = (g0, g1)
        isems = (i0, i1)
        c = lax.axis_index("c")
        s = lax.axis_index("s")
        r0 = s * RPT
        last = BLOCKS - 1

        def wait_idx(slot):
            pltpu.make_async_copy(srcb.at[s, 0], sall.at[slot],
                                  isems[slot]).wait()
            pltpu.make_async_copy(dstb.at[s, 0], dall.at[slot],
                                  isems[slot]).wait()

        def block(slot, other):
            for j in range(IDXB):
                b = j % 2
                pltpu.make_async_copy(table.at[sall.at[slot, j]],
                                      rows.at[b], gsems[b]).wait()
                pltpu.sync_copy(rows.at[b], acc.at[dall.at[slot, j]],
                                add=True)
                if j == IDXB - 2:
                    wait_idx(other)
                jj = j + 2
                if jj < IDXB:
                    idx = sall.at[slot, jj]
                else:
                    idx = sall.at[other, jj - IDXB]
                pltpu.async_copy(table.at[idx], rows.at[b], gsems[b])

        def pair(i, carry):
            block(0, 1)
            nxt = jnp.minimum(2 * i + 2, last)
            pltpu.async_copy(srcb.at[s, nxt], sall.at[0], isems[0])
            pltpu.async_copy(dstb.at[s, nxt], dall.at[0], isems[0])
            block(1, 0)
            nxt2 = jnp.minimum(2 * i + 3, last)
            pltpu.async_copy(srcb.at[s, nxt2], sall.at[1], isems[1])
            pltpu.async_copy(dstb.at[s, nxt2], dall.at[1], isems[1])
            return carry

        for p in range(passes):
            q = c * passes + p
            pltpu.sync_copy(srcb.at[s, 0], sall.at[0])
            pltpu.sync_copy(dstb.at[s, 0], dall.at[0])
            pltpu.async_copy(srcb.at[s, 1], sall.at[1], isems[1])
            pltpu.async_copy(dstb.at[s, 1], dall.at[1], isems[1])
            pltpu.sync_copy(ytab.at[q, pl.ds(r0, RPT)],
                            table.at[pl.ds(r0, RPT)])
            pltpu.sync_copy(ytab.at[q, pl.ds(r0, RPT)],
                            acc.at[pl.ds(r0, RPT)])
            plsc.subcore_barrier()

            for b in range(2):
                pltpu.async_copy(table.at[sall.at[0, b]], rows.at[b],
                                 gsems[b])

            lax.fori_loop(0, BLOCKS // 2, pair, 0)

            for b in range(2):
                pltpu.make_async_copy(table.at[sall.at[0, b]], rows.at[b],
                                      gsems[b]).wait()
            wait_idx(1)

            plsc.subcore_barrier()
            pltpu.sync_copy(acc.at[pl.ds(r0, RPT)],
                            out.at[q, pl.ds(r0, RPT)])

    return agg


_agg64 = _make_agg(64, 2)
_agg80 = _make_agg(80, 1)


def _tc1(xp, rwp, acp, hist, net_W, net_b, g1Wh, g1Wr, g1Wa):
    def body(x_ref, rw_ref, ac_ref, h_ref, nw_ref, nb_ref, wh_ref, wr_ref,
             wa_ref, y1_ref):
        h = jnp.maximum(x_ref[...] @ nw_ref[...] + nb_ref[...], 0.0)
        xw = h @ wh_ref[...] + rw_ref[...] * wr_ref[...] + ac_ref[...] * wa_ref[...]
        deg = 1.0 + jnp.sum(h_ref[...], axis=0)
        dis = 1.0 / jnp.sqrt(deg)
        y = xw * dis[:, None]
        for q in range(4):
            y1_ref[q] = y[:, 64 * q:64 * q + 64]

    return pl.pallas_call(
        body,
        grid=(GRID,),
        in_specs=[
            pl.BlockSpec((HB, 128), lambda i: (i, 0)),
            pl.BlockSpec((HB, 1), lambda i: (i, 0)),
            pl.BlockSpec((HB, 1), lambda i: (i, 0)),
            pl.BlockSpec((NW, HB), lambda i: (0, i)),
            pl.BlockSpec((128, 128), lambda i: (0, 0)),
            pl.BlockSpec((1, 128), lambda i: (0, 0)),
            pl.BlockSpec((128, 256), lambda i: (0, 0)),
            pl.BlockSpec((1, 256), lambda i: (0, 0)),
            pl.BlockSpec((1, 256), lambda i: (0, 0)),
        ],
        out_specs=pl.BlockSpec((4, HB, 64), lambda i: (0, i, 0)),
        out_shape=jax.ShapeDtypeStruct((4, NPAD, 64), jnp.float32),
    )(xp, rwp, acp, hist, net_W, net_b, g1Wh, g1Wr, g1Wa)


def _tc2(agg1, hist, g1b, hv_W, hv_b, epsp, g2Wp):
    def body(a_ref, h_ref, b1_ref, hw_ref, hb_ref, e_ref, w2_ref,
             y2_ref):
        deg = 1.0 + jnp.sum(h_ref[...], axis=0)
        dis = 1.0 / jnp.sqrt(deg)
        agg = jnp.concatenate([a_ref[q] for q in range(4)], axis=1)
        out1 = jnp.maximum(agg * dis[:, None] + b1_ref[...], 0.0)
        x3 = jnp.maximum(out1 @ hw_ref[...] + hb_ref[...], 0.0)
        mean, logvar = x3[:, :128], x3[:, 128:]
        hidden = mean + e_ref[...] * jnp.exp(jnp.clip(logvar, -5.0, 5.0))
        y2 = (hidden @ w2_ref[...]) * dis[:, None]
        y2_ref[0] = y2[:, :80]
        y2_ref[1] = y2[:, 80:]

    return pl.pallas_call(
        body,
        grid=(GRID,),
        in_specs=[
            pl.BlockSpec((4, HB, 64), lambda i: (0, i, 0)),
            pl.BlockSpec((NW, HB), lambda i: (0, i)),
            pl.BlockSpec((1, 256), lambda i: (0, 0)),
            pl.BlockSpec((256, 256), lambda i: (0, 0)),
            pl.BlockSpec((1, 256), lambda i: (0, 0)),
            pl.BlockSpec((HB, 128), lambda i: (i, 0)),
            pl.BlockSpec((128, 160), lambda i: (0, 0)),
        ],
        out_specs=pl.BlockSpec((2, HB, 80), lambda i: (0, i, 0)),
        out_shape=jax.ShapeDtypeStruct((2, NPAD, 80), jnp.float32),
    )(agg1, hist, g1b, hv_W, hv_b, epsp, g2Wp)


def _tc3(agg2, hist, g2b):
    def body(a_ref, h_ref, b2_ref, o_ref):
        deg = 1.0 + jnp.sum(h_ref[...], axis=0)
        dis = 1.0 / jnp.sqrt(deg)
        agg = jnp.concatenate([a_ref[0], a_ref[1]], axis=1)
        o_ref[...] = jnp.maximum(agg * dis[:, None] + b2_ref[...], 0.0)

    return pl.pallas_call(
        body,
        grid=(GRID,),
        in_specs=[
            pl.BlockSpec((2, HB, 80), lambda i: (0, i, 0)),
            pl.BlockSpec((NW, HB), lambda i: (0, i)),
            pl.BlockSpec((1, 160), lambda i: (0, 0)),
        ],
        out_specs=pl.BlockSpec((HB, 160), lambda i: (i, 0)),
        out_shape=jax.ShapeDtypeStruct((NPAD, 160), jnp.float32),
    )(agg2, hist, g2b)


def kernel(x, rw1, action, edge_index, net_W, net_b, gcn1_W, gcn1_b, hv_W,
           hv_b, gcn2_W, gcn2_b):
    src, dst = edge_index[0], edge_index[1]
    pad = jnp.full((EPAD - E,), N, dtype=src.dtype)
    srcb = jnp.concatenate([src, pad]).reshape(NS, BLOCKS, IDXB, KE)
    dstb = jnp.concatenate([dst, pad]).reshape(NS, BLOCKS, IDXB, KE)

    xp = jnp.pad(x, ((0, NPAD - N), (0, 0)))
    rwp = jnp.pad(rw1, ((0, NPAD - N), (0, 0)))
    acp = jnp.pad(action, ((0, NPAD - N), (0, 0)))
    eps = jax.random.normal(jax.random.key(42), (N, 128), dtype=jnp.float32)
    epsp = jnp.pad(eps, ((0, NPAD - N), (0, 0)))

    g1Wh = gcn1_W[:128]
    g1Wr = gcn1_W[128:129]
    g1Wa = gcn1_W[129:130]
    g2Wp = jnp.pad(gcn2_W, ((0, 0), (0, 31)))
    g2bp = jnp.pad(gcn2_b, (0, 31))

    hist = _hist_kernel(dst)
    y1tab = _tc1(xp, rwp, acp, hist, net_W, net_b[None, :], g1Wh, g1Wr, g1Wa)
    agg1 = _agg64(y1tab, srcb, dstb)
    y2tab = _tc2(agg1, hist, gcn1_b[None, :], hv_W, hv_b[None, :],
                 epsp, g2Wp)
    agg2 = _agg80(y2tab, srcb, dstb)
    res = _tc3(agg2, hist, g2bp[None, :])
    r = res[:N]
    return (r[:, :128], r[:, 128])


# speedup vs baseline: 18.5454x; 1.0120x over previous
import functools

import jax
import jax.numpy as jnp
import numpy as np
from jax import lax
from jax.experimental import pallas as pl
from jax.experimental.pallas import tpu as pltpu
from jax.experimental.pallas import tpu_sc as plsc

N = 10000
E = 320000
NC, NS, LANES = 2, 16, 16
NW = NC * NS
NPAD = 10240
EPT = 20480
EPAD = EPT * NS
KE = 128
RPT = NPAD // NS
HB = 1024
GRID = NPAD // HB

_HCH = 2000
_EPW = E // NW


def _hist_kernel(dst):
    mesh = plsc.VectorSubcoreMesh(core_axis_name="c", subcore_axis_name="s")

    @functools.partial(
        pl.kernel,
        out_type=jax.ShapeDtypeStruct((NW, NPAD), jnp.float32),
        mesh=mesh,
        scratch_types=[
            pltpu.VMEM((_HCH,), jnp.int32),
            pltpu.VMEM((NPAD,), jnp.float32),
        ],
        compiler_params=pltpu.CompilerParams(needs_layout_passes=False),
    )
    def hist(dst_hbm, out_hbm, dbuf, hacc):
        c = lax.axis_index("c")
        s = lax.axis_index("s")
        wid = s * NC + c
        zeros = jnp.zeros((LANES,), jnp.float32)
        ones = jnp.ones((LANES,), jnp.float32)

        def zbody(j, carry):
            hacc[pl.ds(j * LANES, LANES)] = zeros
            return carry

        lax.fori_loop(0, NPAD // LANES, zbody, 0)

        for outer in range(_EPW // _HCH):
            base = wid * _EPW + outer * _HCH
            pltpu.sync_copy(dst_hbm.at[pl.ds(base, _HCH)], dbuf)

            def sbody(i, carry):
                v = dbuf[pl.ds(i * LANES, LANES)]
                plsc.addupdate_scatter(hacc, [v], ones)
                return carry

            lax.fori_loop(0, _HCH // LANES, sbody, 0)

        pltpu.sync_copy(hacc, out_hbm.at[wid])

    return hist(dst)


CHUNKS = EPT // KE
IDXB = 8
BLOCKS = CHUNKS // IDXB


def _make_agg(width, passes):
    mesh = plsc.VectorSubcoreMesh(core_axis_name="c", subcore_axis_name="s")

    @functools.partial(
        pl.kernel,
        out_type=jax.ShapeDtypeStruct((NC * passes, NPAD, width),
                                      jnp.float32),
        mesh=mesh,
        scratch_types=[
            pltpu.VMEM_SHARED((NPAD, width), jnp.float32),
            pltpu.VMEM_SHARED((NPAD, width), jnp.float32),
            pltpu.VMEM((2, IDXB, KE), jnp.int32),
            pltpu.VMEM((2, IDXB, KE), jnp.int32),
            pltpu.VMEM((2, KE, width), jnp.float32),
            pltpu.SemaphoreType.DMA,
            pltpu.SemaphoreType.DMA,
            pltpu.SemaphoreType.DMA,
            pltpu.SemaphoreType.DMA,
        ],
        compiler_params=pltpu.CompilerParams(use_tc_tiling_on_sc=False),
    )
    def agg(ytab, srcb, dstb, out, table, acc, sall, dall, rows,
            g0, g1, i0, i1):
        gsems = (g0, g1)
        isems = (i0, i1)
        c = lax.axis_index("c")
        s = lax.axis_index("s")
        r0 = s * RPT
        last = BLOCKS - 1

        def wait_idx(slot):
            pltpu.make_async_copy(srcb.at[s, 0], sall.at[slot],
                                  isems[slot]).wait()
            pltpu.make_async_copy(dstb.at[s, 0], dall.at[slot],
                                  isems[slot]).wait()

        def block(slot, other):
            for j in range(IDXB):
                b = j % 2
                pltpu.make_async_copy(table.at[sall.at[slot, j]],
                                      rows.at[b], gsems[b]).wait()
                pltpu.sync_copy(rows.at[b], acc.at[dall.at[slot, j]],
                                add=True)
                if j == IDXB - 2:
                    wait_idx(other)
                jj = j + 2
                if jj < IDXB:
                    idx = sall.at[slot, jj]
                else:
                    idx = sall.at[other, jj - IDXB]
                pltpu.async_copy(table.at[idx], rows.at[b], gsems[b])

        def pair(i, carry):
            block(0, 1)
            nxt = jnp.minimum(2 * i + 2, last)
            pltpu.async_copy(srcb.at[s, nxt], sall.at[0], isems[0])
            pltpu.async_copy(dstb.at[s, nxt], dall.at[0], isems[0])
            block(1, 0)
            nxt2 = jnp.minimum(2 * i + 3, last)
            pltpu.async_copy(srcb.at[s, nxt2], sall.at[1], isems[1])
            pltpu.async_copy(dstb.at[s, nxt2], dall.at[1], isems[1])
            return carry

        for p in range(passes):
            q = c * passes + p
            pltpu.sync_copy(srcb.at[s, 0], sall.at[0])
            pltpu.sync_copy(dstb.at[s, 0], dall.at[0])
            pltpu.async_copy(srcb.at[s, 1], sall.at[1], isems[1])
            pltpu.async_copy(dstb.at[s, 1], dall.at[1], isems[1])
            pltpu.sync_copy(ytab.at[q, pl.ds(r0, RPT)],
                            table.at[pl.ds(r0, RPT)])
            pltpu.sync_copy(ytab.at[q, pl.ds(r0, RPT)],
                            acc.at[pl.ds(r0, RPT)])
            plsc.subcore_barrier()

            for b in range(2):
                pltpu.async_copy(table.at[sall.at[0, b]], rows.at[b],
                                 gsems[b])

            lax.fori_loop(0, BLOCKS // 2, pair, 0)

            for b in range(2):
                pltpu.make_async_copy(table.at[sall.at[0, b]], rows.at[b],
                                      gsems[b]).wait()
            wait_idx(1)

            plsc.subcore_barrier()
            pltpu.sync_copy(acc.at[pl.ds(r0, RPT)],
                            out.at[q, pl.ds(r0, RPT)])

    return agg


_agg64 = _make_agg(64, 2)
_agg80 = _make_agg(80, 1)


def _tc1(xp, rwp, acp, hist, net_W, net_b, g1Wh, g1Wr, g1Wa):
    def body(x_ref, rw_ref, ac_ref, h_ref, nw_ref, nb_ref, wh_ref, wr_ref,
             wa_ref, y1_ref):
        h = jnp.maximum(x_ref[...] @ nw_ref[...] + nb_ref[...], 0.0)
        xw = h @ wh_ref[...] + rw_ref[...] * wr_ref[...] + ac_ref[...] * wa_ref[...]
        deg = 1.0 + jnp.sum(h_ref[...], axis=0)
        dis = 1.0 / jnp.sqrt(deg)
        y = xw * dis[:, None]
        for q in range(4):
            y1_ref[q] = y[:, 64 * q:64 * q + 64]

    return pl.pallas_call(
        body,
        grid=(GRID,),
        in_specs=[
            pl.BlockSpec((HB, 128), lambda i: (i, 0)),
            pl.BlockSpec((HB, 1), lambda i: (i, 0)),
            pl.BlockSpec((HB, 1), lambda i: (i, 0)),
            pl.BlockSpec((NW, HB), lambda i: (0, i)),
            pl.BlockSpec((128, 128), lambda i: (0, 0)),
            pl.BlockSpec((1, 128), lambda i: (0, 0)),
            pl.BlockSpec((128, 256), lambda i: (0, 0)),
            pl.BlockSpec((1, 256), lambda i: (0, 0)),
            pl.BlockSpec((1, 256), lambda i: (0, 0)),
        ],
        out_specs=pl.BlockSpec((4, HB, 64), lambda i: (0, i, 0)),
        out_shape=jax.ShapeDtypeStruct((4, NPAD, 64), jnp.float32),
    )(xp, rwp, acp, hist, net_W, net_b, g1Wh, g1Wr, g1Wa)


def _tc2(agg1, hist, g1b, hv_W, hv_b, epsp, g2Wp):
    def body(a_ref, h_ref, b1_ref, hw_ref, hb_ref, e_ref, w2_ref,
             y2_ref):
        deg = 1.0 + jnp.sum(h_ref[...], axis=0)
        dis = 1.0 / jnp.sqrt(deg)
        agg = jnp.concatenate([a_ref[q] for q in range(4)], axis=1)
        out1 = jnp.maximum(agg * dis[:, None] + b1_ref[...], 0.0)
        x3 = jnp.maximum(out1 @ hw_ref[...] + hb_ref[...], 0.0)
        mean, logvar = x3[:, :128], x3[:, 128:]
        hidden = mean + e_ref[...] * jnp.exp(jnp.clip(logvar, -5.0, 5.0))
        y2 = (hidden @ w2_ref[...]) * dis[:, None]
        y2_ref[0] = y2[:, :80]
        y2_ref[1] = y2[:, 80:]

    return pl.pallas_call(
        body,
        grid=(GRID,),
        in_specs=[
            pl.BlockSpec((4, HB, 64), lambda i: (0, i, 0)),
            pl.BlockSpec((NW, HB), lambda i: (0, i)),
            pl.BlockSpec((1, 256), lambda i: (0, 0)),
            pl.BlockSpec((256, 256), lambda i: (0, 0)),
            pl.BlockSpec((1, 256), lambda i: (0, 0)),
            pl.BlockSpec((HB, 128), lambda i: (i, 0)),
            pl.BlockSpec((128, 160), lambda i: (0, 0)),
        ],
        out_specs=pl.BlockSpec((2, HB, 80), lambda i: (0, i, 0)),
        out_shape=jax.ShapeDtypeStruct((2, NPAD, 80), jnp.float32),
    )(agg1, hist, g1b, hv_W, hv_b, epsp, g2Wp)


def _tc3(agg2, hist, g2b):
    def body(a_ref, h_ref, b2_ref, o_ref):
        deg = 1.0 + jnp.sum(h_ref[...], axis=0)
        dis = 1.0 / jnp.sqrt(deg)
        agg = jnp.concatenate([a_ref[0], a_ref[1]], axis=1)
        o_ref[...] = jnp.maximum(agg * dis[:, None] + b2_ref[...], 0.0)

    return pl.pallas_call(
        body,
        grid=(GRID,),
        in_specs=[
            pl.BlockSpec((2, HB, 80), lambda i: (0, i, 0)),
            pl.BlockSpec((NW, HB), lambda i: (0, i)),
            pl.BlockSpec((1, 160), lambda i: (0, 0)),
        ],
        out_specs=pl.BlockSpec((HB, 160), lambda i: (i, 0)),
        out_shape=jax.ShapeDtypeStruct((NPAD, 160), jnp.float32),
    )(agg2, hist, g2b)


@functools.cache
def _eps_const():
    with jax.ensure_compile_time_eval():
        eps = jax.random.normal(jax.random.key(42), (N, 128),
                                dtype=jnp.float32)
        return jnp.pad(eps, ((0, NPAD - N), (0, 0)))


def kernel(x, rw1, action, edge_index, net_W, net_b, gcn1_W, gcn1_b, hv_W,
           hv_b, gcn2_W, gcn2_b):
    src, dst = edge_index[0], edge_index[1]
    pad = jnp.full((EPAD - E,), N, dtype=src.dtype)
    srcb = jnp.concatenate([src, pad]).reshape(NS, BLOCKS, IDXB, KE)
    dstb = jnp.concatenate([dst, pad]).reshape(NS, BLOCKS, IDXB, KE)

    epsp = _eps_const()

    g1Wh = gcn1_W[:128]
    g1Wr = gcn1_W[128:129]
    g1Wa = gcn1_W[129:130]
    g2Wp = jnp.pad(gcn2_W, ((0, 0), (0, 31)))
    g2bp = jnp.pad(gcn2_b, (0, 31))

    hist = _hist_kernel(dst)
    y1tab = _tc1(x, rw1, action, hist, net_W, net_b[None, :], g1Wh, g1Wr,
                 g1Wa)
    agg1 = _agg64(y1tab, srcb, dstb)
    y2tab = _tc2(agg1, hist, gcn1_b[None, :], hv_W, hv_b[None, :],
                 epsp, g2Wp)
    agg2 = _agg80(y2tab, srcb, dstb)
    res = _tc3(agg2, hist, g2bp[None, :])
    r = res[:N]
    return (r[:, :128], r[:, 128])
